# Initial kernel scaffold; baseline (speedup 1.0000x reference)
import jax
import jax.numpy as jnp
from jax.experimental import pallas as pl


def kernel(orbpair_hopping, orbpair_onsite, edge_index, edge_cell_shift, kpoints, atom_type):
    raise NotImplementedError("write your pallas kernel here")


# speedup vs baseline: 7.5923x; 7.5923x over previous
import jax
import jax.numpy as jnp
import numpy as np
from jax import lax
from jax.experimental import pallas as pl
from jax.experimental.pallas import tpu as pltpu
from jax.experimental.pallas import tpu_sc as plsc

_LS = [0, 0, 1, 2]
_NORB = 10
_N_ATOMS = 256
_N_EDGES = 4096
_N_KPTS = 4
_FEAT = 68
_ROWW = 32
_BLK = _NORB * _ROWW
_TBW = 384
_OUTW = 2 * _NORB * _N_ATOMS
_BUFW = 5136
_BUFSZ = _NORB * _BUFW + 16
_CH = 32
_SC_CORES = 2
_SC_SUBCORES = 16
_ATOMS_PER_TILE = 8
_EDGE_BITS = 12
_LISTN = _N_EDGES + 256
_EB = 8
_EBS = _N_EDGES // _EB
_NSTART = 272


def _placement_mats():
    off_l = np.cumsum([0] + [2 * l + 1 for l in _LS])[:-1]
    m = np.zeros((_FEAT, 100), np.float32)
    mt = np.zeros((_FEAT, 100), np.float32)
    off = 0
    for i, li in enumerate(_LS):
        for j, lj in enumerate(_LS):
            if i > j:
                continue
            ni, nj = 2 * li + 1, 2 * lj + 1
            fac = 0.5 if i == j else 1.0
            for a in range(ni):
                for b in range(nj):
                    r, c = off_l[i] + a, off_l[j] + b
                    m[off + a * nj + b, r * 10 + c] = fac
                    mt[off + a * nj + b, c * 10 + r] = fac
            off += ni * nj
    def dup(x):
        d = np.zeros((_FEAT, _BLK), np.float32)
        for r in range(10):
            for c in range(10):
                d[:, r * _ROWW + 2 * c] = x[:, r * 10 + c]
                d[:, r * _ROWW + 2 * c + 1] = x[:, r * 10 + c]
        return d

    def dup_re_only(x):
        d = np.zeros((_FEAT, _BLK), np.float32)
        for r in range(10):
            for c in range(10):
                d[:, r * _ROWW + 2 * c] = x[:, r * 10 + c]
        return d

    return dup(m), dup(mt), dup_re_only(m + mt)


_MD, _MDT, _MO = _placement_mats()


def _sorted_list(own_row, own_col, partner_col, ones_row, ones_col):
    e_row = lax.broadcasted_iota(jnp.int32, (1, _N_EDGES), 1).astype(jnp.float32)
    e_col = lax.broadcasted_iota(jnp.int32, (_N_EDGES, 1), 0).astype(jnp.float32)
    keys_row = own_row * _N_EDGES + e_row
    keys_col = own_col * _N_EDGES + e_col
    e_hi = jnp.floor(e_col * (1.0 / 64.0))
    e_lo = e_col - e_hi * 64.0
    vals3 = jnp.concatenate([partner_col, e_hi, e_lo], axis=1)
    gpos_parts = []
    for b in range(_EB):
        krb = keys_row[:, b * _EBS:(b + 1) * _EBS]
        lt = (keys_col < krb).astype(jnp.float32)
        gpos_parts.append(
            jnp.dot(ones_row, lt, preferred_element_type=jnp.float32))
    gpos_row = jnp.concatenate(gpos_parts, axis=1)
    out_parts = []
    for b in range(_EB):
        p_iota = b * _EBS + lax.broadcasted_iota(jnp.int32, (_EBS, 1), 0).astype(jnp.float32)
        oh = (gpos_row == p_iota).astype(jnp.float32)
        v3 = jnp.dot(oh, vals3, preferred_element_type=jnp.float32)
        out_parts.append(v3[:, 0:1] * _N_EDGES + v3[:, 1:2] * 64.0 + v3[:, 2:3])
    lst = jnp.concatenate(
        out_parts + [jnp.zeros((_LISTN - _N_EDGES, 1), jnp.float32)], axis=0)
    a_iota = lax.broadcasted_iota(jnp.int32, (_NSTART, 1), 0).astype(jnp.float32)
    cmp = (own_row < a_iota).astype(jnp.float32)
    starts = jnp.dot(cmp, ones_col, preferred_element_type=jnp.float32)
    return lst, starts


def _tc_prep(hop_ref, ons_ref, shift_ref, k64_ref, src_ref, dst_ref,
             srcc_ref, dstc_ref,
             md_ref, mdt_ref, mo_ref,
             d_ref, dt_ref, onsb_ref, listf_ref, startf_ref,
             listr_ref, startr_ref):
    hop = hop_ref[...]
    onsb_ref[...] = jnp.dot(ons_ref[...], mo_ref[...],
                            preferred_element_type=jnp.float32,
                precision=lax.Precision.HIGHEST)
    t64 = (2.0 * jnp.pi) * jnp.dot(shift_ref[...], k64_ref[...],
                                   preferred_element_type=jnp.float32)
    par = lax.broadcasted_iota(jnp.int32, t64.shape, 1) % 2
    ph = jnp.where(par == 0, jnp.cos(t64), -jnp.sin(t64))

    d_ref[...] = jnp.concatenate(
        [jnp.dot(hop, md_ref[...], preferred_element_type=jnp.float32,
                precision=lax.Precision.HIGHEST), ph],
        axis=1)
    dt_ref[...] = jnp.concatenate(
        [jnp.dot(hop, mdt_ref[...], preferred_element_type=jnp.float32,
                precision=lax.Precision.HIGHEST), ph],
        axis=1)
    ones_row = jnp.ones((1, _N_EDGES), jnp.float32)
    ones_col = jnp.ones((_N_EDGES, 1), jnp.float32)
    src_f = src_ref[...].astype(jnp.float32)
    dst_f = dst_ref[...].astype(jnp.float32)
    srcc_f = srcc_ref[...].astype(jnp.float32)
    dstc_f = dstc_ref[...].astype(jnp.float32)
    lf, sf = _sorted_list(src_f, srcc_f, dstc_f, ones_row, ones_col)
    lr, sr = _sorted_list(dst_f, dstc_f, srcc_f, ones_row, ones_col)
    listf_ref[...] = lf.astype(jnp.int32)
    startf_ref[...] = sf.astype(jnp.int32)
    listr_ref[...] = lr.astype(jnp.int32)
    startr_ref[...] = sr.astype(jnp.int32)


def _sc_body(d_hbm, dt_hbm, onsb_hbm, listf_hbm, startf_hbm,
             listr_hbm, startr_hbm, out_hbm,
             listf, listr, startf, startr, idxbuf, buf, dchunk,
             onsc, sem1):
    wid = lax.axis_index("s") * _SC_CORES + lax.axis_index("c")

    lanes = lax.iota(jnp.int32, 16)
    odd = (lanes % 2) == 1
    altsign = jnp.where(odd, jnp.float32(-1.0), jnp.float32(1.0))

    pltpu.sync_copy(listf_hbm, listf)
    pltpu.sync_copy(listr_hbm, listr)
    pltpu.sync_copy(startf_hbm, startf)
    pltpu.sync_copy(startr_hbm, startr)

    def per_atom(a_local, _):
        a_g = wid * _ATOMS_PER_TILE + a_local
        s0f = startf[pl.ds(a_g, 16)][0]
        s1f = startf[pl.ds(a_g + 1, 16)][0]
        s0r = startr[pl.ds(a_g, 16)][0]
        s1r = startr[pl.ds(a_g + 1, 16)][0]
        nf = s1f - s0f
        nr = s1r - s0r

        pltpu.sync_copy(onsb_hbm.at[a_g], onsc)

        acol0 = a_g * 20

        def edge_adds(tab_hbm, lst, base, n, k, flip):
            nchunks = (n + (_CH - 1)) // _CH

            def chunk_body(c, _c):
                for j in range(_CH // 16):
                    ent = lst[pl.ds(base + c * _CH + j * 16, 16)]
                    idxbuf[pl.ds(j * 16, 16)] = ent & (_N_EDGES - 1)
                pltpu.async_copy(tab_hbm.at[idxbuf], dchunk, sem1).wait()
                cnt = jnp.minimum(n - c * _CH, _CH)

                def one_edge(e, _e):
                    pcol = lax.shift_right_logical(
                        lst[pl.ds(base + c * _CH + e, 16)][0], _EDGE_BITS)
                    acol = pcol * 20
                    pv = dchunk[e, pl.ds(_BLK + 16 * k, 16)]
                    if flip:
                        pv = pv * altsign
                    for i in range(_NORB):
                        woff = i * _BUFW + acol
                        v1 = dchunk[e, pl.ds(i * _ROWW, 16)] * pv
                        plsc.addupdate(buf.at[pl.ds(woff, 16)], v1)
                        v2 = dchunk[e, pl.ds(i * _ROWW + 16, 16)] * pv
                        plsc.addupdate(buf.at[pl.ds(woff + 16, 16)], v2)
                    return _e

                return lax.fori_loop(0, cnt, one_edge, _c)

            lax.fori_loop(0, nchunks, chunk_body, 0)

        def per_k(k, _k):
            def zbody(z, _z):
                buf[pl.ds(z * 16, 16)] = jnp.zeros((16,), jnp.float32)
                return _z
            lax.fori_loop(0, _BUFSZ // 16, zbody, 0)

            for i in range(_NORB):
                woff = i * _BUFW + acol0
                plsc.addupdate(buf.at[pl.ds(woff, 16)],
                               onsc[pl.ds(i * _ROWW, 16)])
                plsc.addupdate(buf.at[pl.ds(woff + 16, 16)],
                               onsc[pl.ds(i * _ROWW + 16, 16)])

            edge_adds(d_hbm, listf, s0f, nf, k, False)
            edge_adds(dt_hbm, listr, s0r, nr, k, True)

            rowbase = (k * (_N_ATOMS * _NORB) + a_g * _NORB) * _OUTW
            for i in range(_NORB):
                pltpu.sync_copy(buf.at[pl.ds(i * _BUFW, _OUTW)],
                                out_hbm.at[pl.ds(rowbase + i * _OUTW, _OUTW)])
            return _k

        lax.fori_loop(0, _N_KPTS, per_k, 0)
        return _

    lax.fori_loop(0, _ATOMS_PER_TILE, per_atom, 0)


@jax.jit
def kernel(orbpair_hopping, orbpair_onsite, edge_index, edge_cell_shift,
           kpoints, atom_type):
    del atom_type
    f32 = jnp.float32
    k64 = jnp.repeat(kpoints.T.astype(f32), 16, axis=1)
    src = edge_index[0].astype(jnp.int32).reshape(1, _N_EDGES)
    dst = edge_index[1].astype(jnp.int32).reshape(1, _N_EDGES)
    srcc = edge_index[0].astype(jnp.int32).reshape(_N_EDGES, 1)
    dstc = edge_index[1].astype(jnp.int32).reshape(_N_EDGES, 1)
    d, dt, onsb, listf, startf, listr, startr = pl.pallas_call(
        _tc_prep,
        out_shape=[
            jax.ShapeDtypeStruct((_N_EDGES, _TBW), f32),
            jax.ShapeDtypeStruct((_N_EDGES, _TBW), f32),
            jax.ShapeDtypeStruct((_N_ATOMS, _BLK), f32),
            jax.ShapeDtypeStruct((_LISTN, 1), jnp.int32),
            jax.ShapeDtypeStruct((_NSTART, 1), jnp.int32),
            jax.ShapeDtypeStruct((_LISTN, 1), jnp.int32),
            jax.ShapeDtypeStruct((_NSTART, 1), jnp.int32),
        ],
    )(orbpair_hopping.astype(f32), orbpair_onsite.astype(f32),
      edge_cell_shift.astype(f32), k64, src, dst, srcc, dstc,
      jnp.asarray(_MD), jnp.asarray(_MDT), jnp.asarray(_MO))

    mesh = plsc.VectorSubcoreMesh(core_axis_name="c", subcore_axis_name="s",
                                  num_cores=_SC_CORES, num_subcores=_SC_SUBCORES)
    all_rows = _N_KPTS * _N_ATOMS * _NORB
    sc = pl.kernel(
        _sc_body,
        out_type=jax.ShapeDtypeStruct((all_rows * _OUTW,), f32),
        mesh=mesh,
        scratch_types=[
            pltpu.VMEM((_LISTN,), jnp.int32),
            pltpu.VMEM((_LISTN,), jnp.int32),
            pltpu.VMEM((_NSTART,), jnp.int32),
            pltpu.VMEM((_NSTART,), jnp.int32),
            pltpu.VMEM((_CH,), jnp.int32),
            pltpu.VMEM((_BUFSZ,), f32),
            pltpu.VMEM((_CH, _TBW), f32),
            pltpu.VMEM((_BLK,), f32),
            pltpu.SemaphoreType.DMA,
        ],
    )
    flat = sc(d, dt, onsb,
              listf.reshape(_LISTN), startf.reshape(_NSTART),
              listr.reshape(_LISTN), startr.reshape(_NSTART))
    out = flat.reshape(_N_KPTS, _N_ATOMS * _NORB, _OUTW).view(jnp.complex64)
    return out
